# baseline (speedup 1.0000x reference)
import functools

import jax
import jax.numpy as jnp
from jax import lax
from jax.experimental import pallas as pl
from jax.experimental.pallas import tpu as pltpu
from jax.experimental.pallas import tpu_sc as plsc

_CH = 128
_TPC = 8
_L = 16


@functools.cache
def _build(B0, K, V, D, NC, NS):
    T2 = D // 8
    T0 = B0 // _CH
    NCHUNK = T0 // _TPC
    mesh = plsc.VectorSubcoreMesh(core_axis_name="c", subcore_axis_name="s")

    @functools.partial(
        pl.kernel,
        mesh=mesh,
        compiler_params=pltpu.CompilerParams(
            use_tc_tiling_on_sc=False, needs_layout_passes=False
        ),
        out_type=jax.ShapeDtypeStruct((K, T2, T0, 8, _CH), jnp.float32),
        scratch_types=[
            pltpu.VMEM((B0,), jnp.int32),
            pltpu.VMEM((2, _CH, D), jnp.float32),
            pltpu.VMEM((T2, _TPC, 8, _CH), jnp.float32),
            pltpu.VMEM((T2, _TPC, 8, _CH), jnp.float32),
            pltpu.SemaphoreType.DMA,
            pltpu.SemaphoreType.DMA,
            pltpu.SemaphoreType.DMA,
            pltpu.SemaphoreType.DMA,
        ],
    )
    def gather(idxT_hbm, table_hbm, out_hbm, idx_v, g_v, s0_v, s1_v,
               gsem0, gsem1, wsem0, wsem1):
        wid = lax.axis_index("s") * NC + lax.axis_index("c")
        gsem = (gsem0, gsem1)
        strips = (s0_v, s1_v)
        wsems = (wsem0, wsem1)

        @pl.when(wid < K)
        def _():
            pltpu.sync_copy(idxT_hbm.at[wid], idx_v)

            iota = lax.iota(jnp.int32, _L)
            t2v = (iota >> 3, (iota >> 3) + 2)
            jiv = iota & 7

            def fire_gather(t0, p):
                pltpu.async_copy(
                    table_hbm.at[idx_v.at[pl.ds(t0 * _CH, _CH)]],
                    g_v.at[p], gsem[p],
                )

            def wait_gather(p):
                pltpu.make_async_copy(
                    table_hbm.at[pl.ds(0, _CH)], g_v.at[p], gsem[p]
                ).wait()

            def transpose_chunk(p, strip, tc):
                tcv = jnp.full((_L,), tc, jnp.int32)

                def row(r, carry):
                    rv = jnp.full((_L,), r, jnp.int32)
                    v0 = g_v[p, r, pl.ds(0, _L)]
                    v1 = g_v[p, r, pl.ds(_L, _L)]
                    plsc.store_scatter(strip, [t2v[0], tcv, jiv, rv], v0)
                    plsc.store_scatter(strip, [t2v[1], tcv, jiv, rv], v1)
                    return carry

                lax.fori_loop(0, _CH, row, 0)

            def fire_writes(c, sq):
                for t2 in range(T2):
                    pltpu.async_copy(
                        strips[sq].at[t2],
                        out_hbm.at[wid, t2, pl.ds(c * _TPC, _TPC)],
                        wsems[sq],
                    )

            def drain_writes(sq):
                for t2 in range(T2):
                    pltpu.make_async_copy(
                        strips[sq].at[t2],
                        out_hbm.at[wid, t2, pl.ds(0, _TPC)],
                        wsems[sq],
                    ).wait()

            fire_gather(0, 0)

            def super_body(i, carry):
                for sq in range(2):
                    c = 2 * i + sq

                    @pl.when(i > 0)
                    def _():
                        drain_writes(sq)

                    for tc in range(_TPC):
                        t0 = c * _TPC + tc
                        p = tc & 1
                        wait_gather(p)
                        if sq == 1 and tc == _TPC - 1:
                            @pl.when(i < NCHUNK // 2 - 1)
                            def _():
                                fire_gather(t0 + 1, 1 - p)
                        else:
                            fire_gather(t0 + 1, 1 - p)
                        transpose_chunk(p, strips[sq], tc)
                    fire_writes(c, sq)
                return carry

            lax.fori_loop(0, NCHUNK // 2, super_body, 0)
            drain_writes(0)
            drain_writes(1)

    return gather


def kernel(indices, R):
    B0, K = indices.shape
    V, D = R.shape
    info = plsc.get_sparse_core_info()
    NC, NS = info.num_cores, info.num_subcores
    idxT = indices.astype(jnp.int32).T
    out5 = _build(B0, K, V, D, NC, NS)(idxT, R)
    t = out5.transpose(2, 4, 0, 1, 3)
    return t.reshape(B0, K, D)
